# baseline (speedup 1.0000x reference)
import functools

import jax
import jax.numpy as jnp
from jax import lax
from jax.experimental import pallas as pl
from jax.experimental.pallas import tpu as pltpu
from jax.experimental.pallas import tpu_sc as plsc

K = 16
B = 8
N = 4096
QB = 256
ROWS = B * N
W = 64
CLAMP = W - 16

NCORES = 2
NSUB = 16
NW = NCORES * NSUB
RPW = ROWS // NW
GROUPS = RPW // 8


def _dist_tau_block(xq_ref, xr_ref, d2_ref, tau_ref):
    xq = xq_ref[0]
    xr = xr_ref[0]
    sq_q = jnp.sum(xq * xq, axis=1)
    sq_r = jnp.sum(xr * xr, axis=1)
    inner = lax.dot_general(
        xq, xr, (((1,), (1,)), ((), ())),
        preferred_element_type=jnp.float32)
    d2 = jnp.maximum((sq_q[:, None] + sq_r[None, :]) - 2.0 * inner, 0.0)
    d2_ref[...] = d2

    sm = d2[:, :128]
    for g in range(1, N // 128):
        sm = jnp.minimum(sm, d2[:, g * 128:(g + 1) * 128])
    inf = jnp.float32(jnp.inf)
    for _ in range(K - 1):
        m = jnp.min(sm, axis=1, keepdims=True)
        sm = jnp.where(sm == m, inf, sm)
    tau_ref[...] = jnp.min(sm, axis=1)


def _sc_compact_body(d2_hbm, tau_hbm, cd_hbm, ci_hbm,
                     rb0, rb1, scd0, scd1, sci0, sci1, tau_v, si0, si1):
    wid = lax.axis_index("s") * NCORES + lax.axis_index("c")
    base = wid * RPW
    pltpu.sync_copy(tau_hbm.at[pl.ds(base, RPW)], tau_v.at[pl.ds(0, RPW)])
    pltpu.async_copy(d2_hbm.at[pl.ds(base, 8)], rb0, si0)
    pltpu.async_copy(d2_hbm.at[pl.ds(base + 8, 8)], rb1, si1)
    iota16 = lax.iota(jnp.int32, 16)
    infv = jnp.full((16,), jnp.inf, jnp.float32)

    def outer(ii, carry):
        for p in range(2):
            rb = (rb0, rb1)[p]
            scd = (scd0, scd1)[p]
            sci = (sci0, sci1)[p]
            si = (si0, si1)[p]
            g = ii * 2 + p
            pltpu.make_async_copy(
                d2_hbm.at[pl.ds(base + g * 8, 8)], rb, si).wait()
            tau_g = tau_v[pl.ds(g * 8, 16)]
            for s in range(8):
                tau_s = tau_g[s]
                for q in range(W // 16):
                    scd[pl.ds(s * W + q * 16, 16)] = infv

                def chunks(jj, off, s=s, rb=rb, scd=scd, sci=sci,
                           tau_s=tau_s):
                    for u in range(4):
                        cbase = jj * 64 + u * 16
                        d = rb[s, pl.ds(cbase, 16)].reshape(16)
                        mask = d <= tau_s
                        idxv = iota16 + cbase
                        ones = jnp.where(mask, 1, 0).astype(jnp.int32)
                        pos = plsc.cumsum(ones) - 1 + off + (s * W)
                        plsc.store_scatter(scd, [pos], d, mask=mask)
                        plsc.store_scatter(sci, [pos], idxv, mask=mask)
                        pc = plsc.all_reduce_population_count(mask)
                        off = jnp.minimum(off + pc, CLAMP)
                    return off

                lax.fori_loop(0, N // 64, chunks,
                              jnp.zeros((16,), jnp.int32))
            pltpu.sync_copy(scd, cd_hbm.at[pl.ds((base + g * 8) * W, 8 * W)])
            pltpu.sync_copy(sci, ci_hbm.at[pl.ds((base + g * 8) * W, 8 * W)])

            @pl.when(g + 2 < GROUPS)
            def _(rb=rb, si=si, g=g):
                pltpu.async_copy(
                    d2_hbm.at[pl.ds(base + (g + 2) * 8, 8)], rb, si)
        return carry

    lax.fori_loop(0, GROUPS // 2, outer, jnp.int32(0))


def _sc_compact(d2, tau):
    mesh = plsc.VectorSubcoreMesh(core_axis_name="c", subcore_axis_name="s")
    f = functools.partial(
        pl.kernel,
        mesh=mesh,
        compiler_params=pltpu.CompilerParams(needs_layout_passes=False),
        out_type=[
            jax.ShapeDtypeStruct((ROWS * W,), jnp.float32),
            jax.ShapeDtypeStruct((ROWS * W,), jnp.int32),
        ],
        scratch_types=[
            pltpu.VMEM((8, N), jnp.float32),
            pltpu.VMEM((8, N), jnp.float32),
            pltpu.VMEM((8 * W,), jnp.float32),
            pltpu.VMEM((8 * W,), jnp.float32),
            pltpu.VMEM((8 * W,), jnp.int32),
            pltpu.VMEM((8 * W,), jnp.int32),
            pltpu.VMEM((RPW + 16,), jnp.float32),
            pltpu.SemaphoreType.DMA,
            pltpu.SemaphoreType.DMA,
        ],
    )(_sc_compact_body)
    return f(d2, tau)


RB2 = 512

def _finish_block(cd_ref, ci_ref, out_ref):
    d = cd_ref[...]
    idx = ci_ref[...]
    big = jnp.int32(N)
    inf = jnp.float32(jnp.inf)
    cols = []
    for _ in range(K):
        m = jnp.min(d, axis=1, keepdims=True)
        masked_idx = jnp.where(d == m, idx, big)
        pick = jnp.min(masked_idx, axis=1)
        cols.append(pick)
        d = jnp.where(idx == pick[:, None], inf, d)
    out_ref[...] = jnp.stack(cols, axis=1)


def kernel(xyz):
    d2, tau = pl.pallas_call(
        _dist_tau_block,
        grid=(B, N // QB),
        in_specs=[
            pl.BlockSpec((1, QB, 3), lambda i, j: (i, j, 0)),
            pl.BlockSpec((1, N, 3), lambda i, j: (i, 0, 0)),
        ],
        out_specs=[
            pl.BlockSpec((QB, N), lambda i, j: (i * (N // QB) + j, 0)),
            pl.BlockSpec((QB,), lambda i, j: (i * (N // QB) + j,)),
        ],
        out_shape=[
            jax.ShapeDtypeStruct((ROWS, N), jnp.float32),
            jax.ShapeDtypeStruct((ROWS,), jnp.float32),
        ],
    )(xyz, xyz)

    cd_flat, ci_flat = _sc_compact(d2, tau)
    cd = cd_flat.reshape(ROWS, W)
    ci = ci_flat.reshape(ROWS, W)

    out = pl.pallas_call(
        _finish_block,
        grid=(ROWS // RB2,),
        in_specs=[
            pl.BlockSpec((RB2, W), lambda i: (i, 0)),
            pl.BlockSpec((RB2, W), lambda i: (i, 0)),
        ],
        out_specs=pl.BlockSpec((RB2, K), lambda i: (i, 0)),
        out_shape=jax.ShapeDtypeStruct((ROWS, K), jnp.int32),
    )(cd, ci)
    return out.reshape(B, N, K).astype(jnp.int64)


# speedup vs baseline: 1.4542x; 1.4542x over previous
import functools

import jax
import jax.numpy as jnp
from jax import lax
from jax.experimental import pallas as pl
from jax.experimental.pallas import tpu as pltpu
from jax.experimental.pallas import tpu_sc as plsc

K = 16
B = 8
N = 4096
QB = 256
ROWS = B * N
SLOTS = 12
W = 16 * SLOTS

NCORES = 2
NSUB = 16
NW = NCORES * NSUB
RPW = ROWS // NW
GROUPS = RPW // 8


def _dist_tau_block(xq_ref, xr_ref, d2_ref, tau_ref):
    xq = xq_ref[0]
    xr = xr_ref[0]
    sq_q = jnp.sum(xq * xq, axis=1)
    sq_r = jnp.sum(xr * xr, axis=1)
    inner = lax.dot_general(
        xq, xr, (((1,), (1,)), ((), ())),
        preferred_element_type=jnp.float32)
    d2 = jnp.maximum((sq_q[:, None] + sq_r[None, :]) - 2.0 * inner, 0.0)
    d2_ref[...] = d2

    sm = d2[:, :128]
    for g in range(1, N // 128):
        sm = jnp.minimum(sm, d2[:, g * 128:(g + 1) * 128])
    inf = jnp.float32(jnp.inf)
    for _ in range(K - 1):
        m = jnp.min(sm, axis=1, keepdims=True)
        sm = jnp.where(sm == m, inf, sm)
    tau_ref[...] = jnp.min(sm, axis=1)


def _sc_compact_body(d2_hbm, tau_hbm, cd_hbm, ci_hbm,
                     rb0, rb1, scd0, scd1, sci0, sci1, tau_v, si0, si1):
    wid = lax.axis_index("s") * NCORES + lax.axis_index("c")
    base = wid * RPW
    pltpu.sync_copy(tau_hbm.at[pl.ds(base, RPW)], tau_v.at[pl.ds(0, RPW)])
    pltpu.async_copy(d2_hbm.at[pl.ds(base, 8)], rb0, si0)
    pltpu.async_copy(d2_hbm.at[pl.ds(base + 8, 8)], rb1, si1)
    iota16 = lax.iota(jnp.int32, 16)
    infv = jnp.full((16,), jnp.inf, jnp.float32)

    def outer(ii, carry):
        for p in range(2):
            rb = (rb0, rb1)[p]
            scd = (scd0, scd1)[p]
            sci = (sci0, sci1)[p]
            si = (si0, si1)[p]
            g = ii * 2 + p
            pltpu.make_async_copy(
                d2_hbm.at[pl.ds(base + g * 8, 8)], rb, si).wait()
            tau_g = tau_v[pl.ds(g * 8, 16)]
            for s in range(8):
                tau_s = tau_g[s]
                for q in range(W // 16):
                    scd[pl.ds(s * W + q * 16, 16)] = infv
                basev = iota16 * SLOTS + (s * W)

                def chunks(jj, cnt, s=s, rb=rb, scd=scd, sci=sci,
                           tau_s=tau_s, basev=basev):
                    for u in range(4):
                        cbase = jj * 64 + u * 16
                        d = rb[s, pl.ds(cbase, 16)].reshape(16)
                        mask = d <= tau_s
                        idxv = iota16 + cbase
                        pos = basev + cnt
                        plsc.store_scatter(scd, [pos], d, mask=mask)
                        plsc.store_scatter(sci, [pos], idxv, mask=mask)
                        cnt = jnp.minimum(
                            cnt + jnp.where(mask, 1, 0).astype(jnp.int32),
                            SLOTS - 1)
                    return cnt

                lax.fori_loop(0, N // 64, chunks,
                              jnp.zeros((16,), jnp.int32))
            pltpu.sync_copy(scd, cd_hbm.at[pl.ds((base + g * 8) * W, 8 * W)])
            pltpu.sync_copy(sci, ci_hbm.at[pl.ds((base + g * 8) * W, 8 * W)])

            @pl.when(g + 2 < GROUPS)
            def _(rb=rb, si=si, g=g):
                pltpu.async_copy(
                    d2_hbm.at[pl.ds(base + (g + 2) * 8, 8)], rb, si)
        return carry

    lax.fori_loop(0, GROUPS // 2, outer, jnp.int32(0))


def _sc_compact(d2, tau):
    mesh = plsc.VectorSubcoreMesh(core_axis_name="c", subcore_axis_name="s")
    f = functools.partial(
        pl.kernel,
        mesh=mesh,
        compiler_params=pltpu.CompilerParams(needs_layout_passes=False),
        out_type=[
            jax.ShapeDtypeStruct((ROWS * W,), jnp.float32),
            jax.ShapeDtypeStruct((ROWS * W,), jnp.int32),
        ],
        scratch_types=[
            pltpu.VMEM((8, N), jnp.float32),
            pltpu.VMEM((8, N), jnp.float32),
            pltpu.VMEM((8 * W,), jnp.float32),
            pltpu.VMEM((8 * W,), jnp.float32),
            pltpu.VMEM((8 * W,), jnp.int32),
            pltpu.VMEM((8 * W,), jnp.int32),
            pltpu.VMEM((RPW + 16,), jnp.float32),
            pltpu.SemaphoreType.DMA,
            pltpu.SemaphoreType.DMA,
        ],
    )(_sc_compact_body)
    return f(d2, tau)


RB2 = 512

def _finish_block(cd_ref, ci_ref, out_ref):
    d = cd_ref[...]
    idx = ci_ref[...]
    big = jnp.int32(N)
    inf = jnp.float32(jnp.inf)
    cols = []
    for _ in range(K):
        m = jnp.min(d, axis=1, keepdims=True)
        masked_idx = jnp.where(d == m, idx, big)
        pick = jnp.min(masked_idx, axis=1)
        cols.append(pick)
        d = jnp.where(idx == pick[:, None], inf, d)
    out_ref[...] = jnp.stack(cols, axis=1)


def kernel(xyz):
    d2, tau = pl.pallas_call(
        _dist_tau_block,
        grid=(B, N // QB),
        in_specs=[
            pl.BlockSpec((1, QB, 3), lambda i, j: (i, j, 0)),
            pl.BlockSpec((1, N, 3), lambda i, j: (i, 0, 0)),
        ],
        out_specs=[
            pl.BlockSpec((QB, N), lambda i, j: (i * (N // QB) + j, 0)),
            pl.BlockSpec((QB,), lambda i, j: (i * (N // QB) + j,)),
        ],
        out_shape=[
            jax.ShapeDtypeStruct((ROWS, N), jnp.float32),
            jax.ShapeDtypeStruct((ROWS,), jnp.float32),
        ],
    )(xyz, xyz)

    cd_flat, ci_flat = _sc_compact(d2, tau)
    cd = cd_flat.reshape(ROWS, W)
    ci = ci_flat.reshape(ROWS, W)

    out = pl.pallas_call(
        _finish_block,
        grid=(ROWS // RB2,),
        in_specs=[
            pl.BlockSpec((RB2, W), lambda i: (i, 0)),
            pl.BlockSpec((RB2, W), lambda i: (i, 0)),
        ],
        out_specs=pl.BlockSpec((RB2, K), lambda i: (i, 0)),
        out_shape=jax.ShapeDtypeStruct((ROWS, K), jnp.int32),
    )(cd, ci)
    return out.reshape(B, N, K).astype(jnp.int64)


# speedup vs baseline: 1.5139x; 1.0411x over previous
import functools

import jax
import jax.numpy as jnp
from jax import lax
from jax.experimental import pallas as pl
from jax.experimental.pallas import tpu as pltpu
from jax.experimental.pallas import tpu_sc as plsc

K = 16
B = 8
N = 4096
QB = 256
ROWS = B * N
SLOTS = 12
W = 16 * SLOTS

NCORES = 2
NSUB = 16
NW = NCORES * NSUB
RPW = ROWS // NW
GROUPS = RPW // 8


def _dist_tau_block(xq_ref, xr_ref, d2_ref, tau_ref):
    xq = xq_ref[0]
    xr = xr_ref[0]
    sq_q = jnp.sum(xq * xq, axis=1)
    sq_r = jnp.sum(xr * xr, axis=1)
    inner = lax.dot_general(
        xq, xr, (((1,), (1,)), ((), ())),
        preferred_element_type=jnp.float32)
    d2 = jnp.maximum((sq_q[:, None] + sq_r[None, :]) - 2.0 * inner, 0.0)
    d2_ref[...] = d2

    sm = d2[:, :128]
    for g in range(1, N // 128):
        sm = jnp.minimum(sm, d2[:, g * 128:(g + 1) * 128])
    inf = jnp.float32(jnp.inf)
    for _ in range(K - 1):
        m = jnp.min(sm, axis=1, keepdims=True)
        sm = jnp.where(sm == m, inf, sm)
    tau_ref[...] = jnp.min(sm, axis=1)


def _sc_compact_body(d2_hbm, tau_hbm, cd_hbm, ci_hbm,
                     rb0, rb1, scd0, scd1, sraw0, sraw1, sci0, sci1,
                     tau_v, si0, si1):
    wid = lax.axis_index("s") * NCORES + lax.axis_index("c")
    base = wid * RPW
    pltpu.sync_copy(tau_hbm.at[pl.ds(base, RPW)], tau_v.at[pl.ds(0, RPW)])
    pltpu.async_copy(d2_hbm.at[pl.ds(base, 8)], rb0, si0)
    pltpu.async_copy(d2_hbm.at[pl.ds(base + 8, 8)], rb1, si1)
    iota16 = lax.iota(jnp.int32, 16)
    infv = jnp.full((16,), jnp.inf, jnp.float32)

    def outer(ii, carry):
        for p in range(2):
            rb = (rb0, rb1)[p]
            scd = (scd0, scd1)[p]
            sraw = (sraw0, sraw1)[p]
            sci = (sci0, sci1)[p]
            si = (si0, si1)[p]
            g = ii * 2 + p
            pltpu.make_async_copy(
                d2_hbm.at[pl.ds(base + g * 8, 8)], rb, si).wait()
            tau_g = tau_v[pl.ds(g * 8, 16)]
            for s in range(8):
                tau_s = tau_g[s]
                zv = jnp.zeros((16,), jnp.int32)
                for q in range(W // 16):
                    sraw[pl.ds(s * W + q * 16, 16)] = zv
                basev = iota16 * SLOTS + (s * W)

                def chunks(jj, cnt, s=s, rb=rb, sraw=sraw,
                           tau_s=tau_s, basev=basev):
                    for u in range(8):
                        cbase = jj * 128 + u * 16
                        d = rb[s, pl.ds(cbase, 16)].reshape(16)
                        mask = d <= tau_s
                        idxv = iota16 + cbase
                        plsc.store_scatter(sraw, [basev + cnt], idxv,
                                           mask=mask)
                        cnt = jnp.minimum(
                            cnt + jnp.where(mask, 1, 0).astype(jnp.int32),
                            SLOTS - 1)
                    return cnt

                cnt = lax.fori_loop(0, N // 128, chunks,
                                    jnp.zeros((16,), jnp.int32))
                sfull = jnp.full((16,), s, jnp.int32)
                for k in range(SLOTS):
                    slotpos = iota16 * SLOTS + (s * W + k)
                    idx_k = plsc.load_gather(sraw, [slotpos])
                    d_k = plsc.load_gather(rb, [sfull, idx_k])
                    filled = cnt > k
                    d_k = jnp.where(filled, d_k, infv)
                    scd[pl.ds(s * W + k * 16, 16)] = d_k
                    sci[pl.ds(s * W + k * 16, 16)] = idx_k
            pltpu.sync_copy(scd, cd_hbm.at[pl.ds((base + g * 8) * W, 8 * W)])
            pltpu.sync_copy(sci, ci_hbm.at[pl.ds((base + g * 8) * W, 8 * W)])

            @pl.when(g + 2 < GROUPS)
            def _(rb=rb, si=si, g=g):
                pltpu.async_copy(
                    d2_hbm.at[pl.ds(base + (g + 2) * 8, 8)], rb, si)
        return carry

    lax.fori_loop(0, GROUPS // 2, outer, jnp.int32(0))


def _sc_compact(d2, tau):
    mesh = plsc.VectorSubcoreMesh(core_axis_name="c", subcore_axis_name="s")
    f = functools.partial(
        pl.kernel,
        mesh=mesh,
        compiler_params=pltpu.CompilerParams(needs_layout_passes=False),
        out_type=[
            jax.ShapeDtypeStruct((ROWS * W,), jnp.float32),
            jax.ShapeDtypeStruct((ROWS * W,), jnp.int32),
        ],
        scratch_types=[
            pltpu.VMEM((8, N), jnp.float32),
            pltpu.VMEM((8, N), jnp.float32),
            pltpu.VMEM((8 * W,), jnp.float32),
            pltpu.VMEM((8 * W,), jnp.float32),
            pltpu.VMEM((8 * W,), jnp.int32),
            pltpu.VMEM((8 * W,), jnp.int32),
            pltpu.VMEM((8 * W,), jnp.int32),
            pltpu.VMEM((8 * W,), jnp.int32),
            pltpu.VMEM((RPW + 16,), jnp.float32),
            pltpu.SemaphoreType.DMA,
            pltpu.SemaphoreType.DMA,
        ],
    )(_sc_compact_body)
    return f(d2, tau)


RB2 = 512

def _finish_block(cd_ref, ci_ref, out_ref):
    d = cd_ref[...]
    idx = ci_ref[...]
    big = jnp.int32(N)
    inf = jnp.float32(jnp.inf)
    cols = []
    for _ in range(K):
        m = jnp.min(d, axis=1, keepdims=True)
        masked_idx = jnp.where(d == m, idx, big)
        pick = jnp.min(masked_idx, axis=1)
        cols.append(pick)
        d = jnp.where(idx == pick[:, None], inf, d)
    out_ref[...] = jnp.stack(cols, axis=1)


def kernel(xyz):
    d2, tau = pl.pallas_call(
        _dist_tau_block,
        grid=(B, N // QB),
        in_specs=[
            pl.BlockSpec((1, QB, 3), lambda i, j: (i, j, 0)),
            pl.BlockSpec((1, N, 3), lambda i, j: (i, 0, 0)),
        ],
        out_specs=[
            pl.BlockSpec((QB, N), lambda i, j: (i * (N // QB) + j, 0)),
            pl.BlockSpec((QB,), lambda i, j: (i * (N // QB) + j,)),
        ],
        out_shape=[
            jax.ShapeDtypeStruct((ROWS, N), jnp.float32),
            jax.ShapeDtypeStruct((ROWS,), jnp.float32),
        ],
    )(xyz, xyz)

    cd_flat, ci_flat = _sc_compact(d2, tau)
    cd = cd_flat.reshape(ROWS, W)
    ci = ci_flat.reshape(ROWS, W)

    out = pl.pallas_call(
        _finish_block,
        grid=(ROWS // RB2,),
        in_specs=[
            pl.BlockSpec((RB2, W), lambda i: (i, 0)),
            pl.BlockSpec((RB2, W), lambda i: (i, 0)),
        ],
        out_specs=pl.BlockSpec((RB2, K), lambda i: (i, 0)),
        out_shape=jax.ShapeDtypeStruct((ROWS, K), jnp.int32),
    )(cd, ci)
    return out.reshape(B, N, K).astype(jnp.int64)


# speedup vs baseline: 1.5254x; 1.0076x over previous
import functools

import jax
import jax.numpy as jnp
from jax import lax
from jax.experimental import pallas as pl
from jax.experimental.pallas import tpu as pltpu
from jax.experimental.pallas import tpu_sc as plsc

K = 16
B = 8
N = 4096
QB = 256
ROWS = B * N
SLOTS = 12
W = 16 * SLOTS

NCORES = 2
NSUB = 16
NW = NCORES * NSUB
RPW = ROWS // NW
GROUPS = RPW // 8


def _dist_tau_block(xq_ref, xr_ref, d2_ref, tau_ref):
    xq = xq_ref[0]
    xr = xr_ref[0]
    sq_q = jnp.sum(xq * xq, axis=1)
    sq_r = jnp.sum(xr * xr, axis=1)
    inner = lax.dot_general(
        xq, xr, (((1,), (1,)), ((), ())),
        preferred_element_type=jnp.float32)
    d2 = jnp.maximum((sq_q[:, None] + sq_r[None, :]) - 2.0 * inner, 0.0)
    d2_ref[...] = d2

    sm = d2[:, :128]
    for g in range(1, N // 128):
        sm = jnp.minimum(sm, d2[:, g * 128:(g + 1) * 128])
    inf = jnp.float32(jnp.inf)
    for _ in range(K - 1):
        m = jnp.min(sm, axis=1, keepdims=True)
        sm = jnp.where(sm == m, inf, sm)
    tau_ref[...] = jnp.min(sm, axis=1)


def _sc_compact_body(d2_hbm, tau_hbm, cd_hbm, ci_hbm,
                     rb0, rb1, scd0, scd1, sraw0, sraw1, sci0, sci1,
                     tau_v, si0, si1):
    wid = lax.axis_index("s") * NCORES + lax.axis_index("c")
    base = wid * RPW
    pltpu.sync_copy(tau_hbm.at[pl.ds(base, RPW)], tau_v.at[pl.ds(0, RPW)])

    def _fetch(grp, rb, si):
        for s in range(8):
            pltpu.async_copy(d2_hbm.at[base + grp * 8 + s],
                             rb.at[pl.ds(s * N, N)], si)

    def _drain(grp, rb, si):
        for s in range(8):
            pltpu.make_async_copy(d2_hbm.at[base + grp * 8 + s],
                                  rb.at[pl.ds(s * N, N)], si).wait()

    _fetch(0, rb0, si0)
    _fetch(1, rb1, si1)
    iota16 = lax.iota(jnp.int32, 16)
    infv = jnp.full((16,), jnp.inf, jnp.float32)

    def outer(ii, carry):
        for p in range(2):
            rb = (rb0, rb1)[p]
            scd = (scd0, scd1)[p]
            sraw = (sraw0, sraw1)[p]
            sci = (sci0, sci1)[p]
            si = (si0, si1)[p]
            g = ii * 2 + p
            _drain(g, rb, si)
            tau_g = tau_v[pl.ds(g * 8, 16)]
            for s in range(8):
                tau_s = tau_g[s]
                zv = jnp.zeros((16,), jnp.int32)
                for q in range(W // 16):
                    sraw[pl.ds(s * W + q * 16, 16)] = zv
                basev = iota16 * SLOTS + (s * W)

                def chunks(jj, cnt, s=s, rb=rb, sraw=sraw,
                           tau_s=tau_s, basev=basev):
                    for u in range(8):
                        cbase = s * N + jj * 128 + u * 16
                        d = rb[pl.ds(cbase, 16)]
                        mask = d <= tau_s
                        idxv = iota16 + (cbase - s * N)
                        plsc.store_scatter(sraw, [basev + cnt], idxv,
                                           mask=mask)
                        cnt = jnp.minimum(
                            cnt + jnp.where(mask, 1, 0).astype(jnp.int32),
                            SLOTS - 1)
                    return cnt

                cnt = lax.fori_loop(0, N // 128, chunks,
                                    jnp.zeros((16,), jnp.int32))
                for k in range(SLOTS):
                    slotpos = iota16 * SLOTS + (s * W + k)
                    idx_k = plsc.load_gather(sraw, [slotpos])
                    d_k = plsc.load_gather(rb, [idx_k + (s * N)])
                    filled = cnt > k
                    d_k = jnp.where(filled, d_k, infv)
                    scd[pl.ds(s * W + k * 16, 16)] = d_k
                    sci[pl.ds(s * W + k * 16, 16)] = idx_k
            pltpu.sync_copy(scd, cd_hbm.at[pl.ds((base + g * 8) * W, 8 * W)])
            pltpu.sync_copy(sci, ci_hbm.at[pl.ds((base + g * 8) * W, 8 * W)])

            @pl.when(g + 2 < GROUPS)
            def _(rb=rb, si=si, g=g):
                _fetch(g + 2, rb, si)
        return carry

    lax.fori_loop(0, GROUPS // 2, outer, jnp.int32(0))


def _sc_compact(d2, tau):
    mesh = plsc.VectorSubcoreMesh(core_axis_name="c", subcore_axis_name="s")
    f = functools.partial(
        pl.kernel,
        mesh=mesh,
        compiler_params=pltpu.CompilerParams(needs_layout_passes=False),
        out_type=[
            jax.ShapeDtypeStruct((ROWS * W,), jnp.float32),
            jax.ShapeDtypeStruct((ROWS * W,), jnp.int32),
        ],
        scratch_types=[
            pltpu.VMEM((8 * N,), jnp.float32),
            pltpu.VMEM((8 * N,), jnp.float32),
            pltpu.VMEM((8 * W,), jnp.float32),
            pltpu.VMEM((8 * W,), jnp.float32),
            pltpu.VMEM((8 * W,), jnp.int32),
            pltpu.VMEM((8 * W,), jnp.int32),
            pltpu.VMEM((8 * W,), jnp.int32),
            pltpu.VMEM((8 * W,), jnp.int32),
            pltpu.VMEM((RPW + 16,), jnp.float32),
            pltpu.SemaphoreType.DMA,
            pltpu.SemaphoreType.DMA,
        ],
    )(_sc_compact_body)
    return f(d2, tau)


RB2 = 512

def _finish_block(cd_ref, ci_ref, out_ref):
    d = cd_ref[...]
    idx = ci_ref[...]
    big = jnp.int32(N)
    inf = jnp.float32(jnp.inf)
    cols = []
    for _ in range(K):
        m = jnp.min(d, axis=1, keepdims=True)
        masked_idx = jnp.where(d == m, idx, big)
        pick = jnp.min(masked_idx, axis=1)
        cols.append(pick)
        d = jnp.where(idx == pick[:, None], inf, d)
    out_ref[...] = jnp.stack(cols, axis=1)


def kernel(xyz):
    d2, tau = pl.pallas_call(
        _dist_tau_block,
        grid=(B, N // QB),
        in_specs=[
            pl.BlockSpec((1, QB, 3), lambda i, j: (i, j, 0)),
            pl.BlockSpec((1, N, 3), lambda i, j: (i, 0, 0)),
        ],
        out_specs=[
            pl.BlockSpec((QB, N), lambda i, j: (i * (N // QB) + j, 0)),
            pl.BlockSpec((QB,), lambda i, j: (i * (N // QB) + j,)),
        ],
        out_shape=[
            jax.ShapeDtypeStruct((ROWS, N), jnp.float32),
            jax.ShapeDtypeStruct((ROWS,), jnp.float32),
        ],
    )(xyz, xyz)

    cd_flat, ci_flat = _sc_compact(d2, tau)
    cd = cd_flat.reshape(ROWS, W)
    ci = ci_flat.reshape(ROWS, W)

    out = pl.pallas_call(
        _finish_block,
        grid=(ROWS // RB2,),
        in_specs=[
            pl.BlockSpec((RB2, W), lambda i: (i, 0)),
            pl.BlockSpec((RB2, W), lambda i: (i, 0)),
        ],
        out_specs=pl.BlockSpec((RB2, K), lambda i: (i, 0)),
        out_shape=jax.ShapeDtypeStruct((ROWS, K), jnp.int32),
    )(cd, ci)
    return out.reshape(B, N, K).astype(jnp.int64)


# speedup vs baseline: 2.7438x; 1.7987x over previous
import functools

import jax
import jax.numpy as jnp
from jax import lax
from jax.experimental import pallas as pl
from jax.experimental.pallas import tpu as pltpu
from jax.experimental.pallas import tpu_sc as plsc

K = 16
B = 8
N = 4096
QB = 256
ROWS = B * N
SLOTS = 12
W = 16 * SLOTS

NCORES = 2
NSUB = 16
NW = NCORES * NSUB
RPW = ROWS // NW
GROUPS = RPW // 8


def _dist_tau_block(xq_ref, xr_ref, d2_ref, tau_ref):
    xq = xq_ref[0]
    xr = xr_ref[0]
    sq_q = jnp.sum(xq * xq, axis=1)
    sq_r = jnp.sum(xr * xr, axis=1)
    inner = lax.dot_general(
        xq, xr, (((1,), (1,)), ((), ())),
        preferred_element_type=jnp.float32)
    d2 = jnp.maximum((sq_q[:, None] + sq_r[None, :]) - 2.0 * inner, 0.0)
    d2_ref[...] = d2

    sm = d2[:, :128]
    for g in range(1, N // 128):
        sm = jnp.minimum(sm, d2[:, g * 128:(g + 1) * 128])
    inf = jnp.float32(jnp.inf)
    for _ in range(K - 1):
        m = jnp.min(sm, axis=1, keepdims=True)
        sm = jnp.where(sm == m, inf, sm)
    tau_ref[...] = jnp.min(sm, axis=1)


def _sc_compact_body(d2_hbm, tau_hbm, cd_hbm, ci_hbm,
                     rb0, rb1, scd0, scd1, sraw0, sraw1, sci0, sci1,
                     tau_v, si0, si1):
    wid = lax.axis_index("s") * NCORES + lax.axis_index("c")
    base = wid * RPW
    pltpu.sync_copy(tau_hbm.at[pl.ds(base, RPW)], tau_v.at[pl.ds(0, RPW)])

    def _fetch(grp, rb, si):
        for s in range(8):
            pltpu.async_copy(d2_hbm.at[base + grp * 8 + s],
                             rb.at[pl.ds(s * N, N)], si)

    def _drain(grp, rb, si):
        for s in range(8):
            pltpu.make_async_copy(d2_hbm.at[base + grp * 8 + s],
                                  rb.at[pl.ds(s * N, N)], si).wait()

    _fetch(0, rb0, si0)
    _fetch(1, rb1, si1)
    iota16 = lax.iota(jnp.int32, 16)
    infv = jnp.full((16,), jnp.inf, jnp.float32)

    def outer(ii, carry):
        for p in range(2):
            rb = (rb0, rb1)[p]
            scd = (scd0, scd1)[p]
            sraw = (sraw0, sraw1)[p]
            sci = (sci0, sci1)[p]
            si = (si0, si1)[p]
            g = ii * 2 + p
            _drain(g, rb, si)
            tau_g = tau_v[pl.ds(g * 8, 16)]
            for s in range(8):
                tau_s = tau_g[s]
                zv = jnp.zeros((16,), jnp.int32)
                for q in range(W // 16):
                    sraw[pl.ds(s * W + q * 16, 16)] = zv
                basev = iota16 * SLOTS + (s * W)

                def chunks(jj, cnt, s=s, rb=rb, sraw=sraw,
                           tau_s=tau_s, basev=basev):
                    masks, ones = [], []
                    for u in range(8):
                        d = rb[pl.ds(s * N + jj * 128 + u * 16, 16)]
                        m = d <= tau_s
                        masks.append(m)
                        ones.append(jnp.where(m, 1, 0).astype(jnp.int32))
                    s01 = ones[0] + ones[1]
                    s23 = ones[2] + ones[3]
                    s45 = ones[4] + ones[5]
                    s67 = ones[6] + ones[7]
                    s03 = s01 + s23
                    s47 = s45 + s67
                    pref = [None, ones[0], s01, s01 + ones[2],
                            s03, s03 + ones[4], s03 + s45,
                            s03 + (s45 + ones[6])]
                    for u in range(8):
                        c_u = cnt if u == 0 else cnt + pref[u]
                        pos = basev + jnp.minimum(c_u, SLOTS - 1)
                        idxv = iota16 + (jj * 128 + u * 16)
                        plsc.store_scatter(sraw, [pos], idxv, mask=masks[u])
                    return jnp.minimum(cnt + (s03 + s47), SLOTS - 1)

                cnt = lax.fori_loop(0, N // 128, chunks,
                                    jnp.zeros((16,), jnp.int32))
                for k in range(SLOTS):
                    slotpos = iota16 * SLOTS + (s * W + k)
                    idx_k = plsc.load_gather(sraw, [slotpos])
                    d_k = plsc.load_gather(rb, [idx_k + (s * N)])
                    filled = cnt > k
                    d_k = jnp.where(filled, d_k, infv)
                    scd[pl.ds(s * W + k * 16, 16)] = d_k
                    sci[pl.ds(s * W + k * 16, 16)] = idx_k
            pltpu.sync_copy(scd, cd_hbm.at[pl.ds((base + g * 8) * W, 8 * W)])
            pltpu.sync_copy(sci, ci_hbm.at[pl.ds((base + g * 8) * W, 8 * W)])

            @pl.when(g + 2 < GROUPS)
            def _(rb=rb, si=si, g=g):
                _fetch(g + 2, rb, si)
        return carry

    lax.fori_loop(0, GROUPS // 2, outer, jnp.int32(0))


def _sc_compact(d2, tau):
    mesh = plsc.VectorSubcoreMesh(core_axis_name="c", subcore_axis_name="s")
    f = functools.partial(
        pl.kernel,
        mesh=mesh,
        compiler_params=pltpu.CompilerParams(needs_layout_passes=False),
        out_type=[
            jax.ShapeDtypeStruct((ROWS * W,), jnp.float32),
            jax.ShapeDtypeStruct((ROWS * W,), jnp.int32),
        ],
        scratch_types=[
            pltpu.VMEM((8 * N,), jnp.float32),
            pltpu.VMEM((8 * N,), jnp.float32),
            pltpu.VMEM((8 * W,), jnp.float32),
            pltpu.VMEM((8 * W,), jnp.float32),
            pltpu.VMEM((8 * W,), jnp.int32),
            pltpu.VMEM((8 * W,), jnp.int32),
            pltpu.VMEM((8 * W,), jnp.int32),
            pltpu.VMEM((8 * W,), jnp.int32),
            pltpu.VMEM((RPW + 16,), jnp.float32),
            pltpu.SemaphoreType.DMA,
            pltpu.SemaphoreType.DMA,
        ],
    )(_sc_compact_body)
    return f(d2, tau)


RB2 = 512

def _finish_block(cd_ref, ci_ref, out_ref):
    d = cd_ref[...]
    idx = ci_ref[...]
    big = jnp.int32(N)
    inf = jnp.float32(jnp.inf)
    cols = []
    for _ in range(K):
        m = jnp.min(d, axis=1, keepdims=True)
        masked_idx = jnp.where(d == m, idx, big)
        pick = jnp.min(masked_idx, axis=1)
        cols.append(pick)
        d = jnp.where(idx == pick[:, None], inf, d)
    out_ref[...] = jnp.stack(cols, axis=1)


def kernel(xyz):
    d2, tau = pl.pallas_call(
        _dist_tau_block,
        grid=(B, N // QB),
        in_specs=[
            pl.BlockSpec((1, QB, 3), lambda i, j: (i, j, 0)),
            pl.BlockSpec((1, N, 3), lambda i, j: (i, 0, 0)),
        ],
        out_specs=[
            pl.BlockSpec((QB, N), lambda i, j: (i * (N // QB) + j, 0)),
            pl.BlockSpec((QB,), lambda i, j: (i * (N // QB) + j,)),
        ],
        out_shape=[
            jax.ShapeDtypeStruct((ROWS, N), jnp.float32),
            jax.ShapeDtypeStruct((ROWS,), jnp.float32),
        ],
    )(xyz, xyz)

    cd_flat, ci_flat = _sc_compact(d2, tau)
    cd = cd_flat.reshape(ROWS, W)
    ci = ci_flat.reshape(ROWS, W)

    out = pl.pallas_call(
        _finish_block,
        grid=(ROWS // RB2,),
        in_specs=[
            pl.BlockSpec((RB2, W), lambda i: (i, 0)),
            pl.BlockSpec((RB2, W), lambda i: (i, 0)),
        ],
        out_specs=pl.BlockSpec((RB2, K), lambda i: (i, 0)),
        out_shape=jax.ShapeDtypeStruct((ROWS, K), jnp.int32),
    )(cd, ci)
    return out.reshape(B, N, K).astype(jnp.int64)


# speedup vs baseline: 2.7969x; 1.0194x over previous
import functools

import jax
import jax.numpy as jnp
from jax import lax
from jax.experimental import pallas as pl
from jax.experimental.pallas import tpu as pltpu
from jax.experimental.pallas import tpu_sc as plsc

K = 16
B = 8
N = 4096
QB = 256
ROWS = B * N
SLOTS = 12
W = 16 * SLOTS

NCORES = 2
NSUB = 16
NW = NCORES * NSUB
RPW = ROWS // NW
GROUPS = RPW // 8


def _dist_tau_block(xq_ref, xr_ref, d2_ref, tau_ref):
    xq = xq_ref[0]
    xr = xr_ref[0]
    sq_q = jnp.sum(xq * xq, axis=1)
    sq_r = jnp.sum(xr * xr, axis=1)
    inner = lax.dot_general(
        xq, xr, (((1,), (1,)), ((), ())),
        preferred_element_type=jnp.float32)
    d2 = jnp.maximum((sq_q[:, None] + sq_r[None, :]) - 2.0 * inner, 0.0)
    d2_ref[...] = d2

    sm = d2[:, :128]
    for g in range(1, N // 128):
        sm = jnp.minimum(sm, d2[:, g * 128:(g + 1) * 128])
    inf = jnp.float32(jnp.inf)
    for _ in range(K - 1):
        m = jnp.min(sm, axis=1, keepdims=True)
        sm = jnp.where(sm == m, inf, sm)
    tau_ref[...] = jnp.min(sm, axis=1)


def _sc_compact_body(d2_hbm, tau_hbm, cd_hbm, ci_hbm,
                     rb0, rb1, scd0, scd1, sraw0, sraw1, sci0, sci1,
                     tau_v, si0, si1):
    wid = lax.axis_index("s") * NCORES + lax.axis_index("c")
    base = wid * RPW
    pltpu.sync_copy(tau_hbm.at[pl.ds(base, RPW)], tau_v.at[pl.ds(0, RPW)])

    def _fetch(grp, rb, si):
        for s in range(8):
            pltpu.async_copy(d2_hbm.at[base + grp * 8 + s],
                             rb.at[pl.ds(s * N, N)], si)

    def _drain(grp, rb, si):
        for s in range(8):
            pltpu.make_async_copy(d2_hbm.at[base + grp * 8 + s],
                                  rb.at[pl.ds(s * N, N)], si).wait()

    _fetch(0, rb0, si0)
    _fetch(1, rb1, si1)
    iota16 = lax.iota(jnp.int32, 16)
    infv = jnp.full((16,), jnp.inf, jnp.float32)

    def outer(ii, carry):
        for p in range(2):
            rb = (rb0, rb1)[p]
            scd = (scd0, scd1)[p]
            sraw = (sraw0, sraw1)[p]
            sci = (sci0, sci1)[p]
            si = (si0, si1)[p]
            g = ii * 2 + p
            _drain(g, rb, si)
            tau_g = tau_v[pl.ds(g * 8, 16)]
            for s in range(8):
                tau_s = tau_g[s]
                zv = jnp.zeros((16,), jnp.int32)
                for q in range(W // 16):
                    sraw[pl.ds(s * W + q * 16, 16)] = zv
                basev = iota16 * SLOTS + (s * W)

                def chunks(jj, cnt, s=s, rb=rb, sraw=sraw,
                           tau_s=tau_s, basev=basev):
                    masks, ones = [], []
                    for u in range(16):
                        d = rb[pl.ds(s * N + jj * 256 + u * 16, 16)]
                        m = d <= tau_s
                        masks.append(m)
                        ones.append(jnp.where(m, 1, 0).astype(jnp.int32))
                    t = [ones[2 * i] + ones[2 * i + 1] for i in range(8)]
                    q = [t[2 * i] + t[2 * i + 1] for i in range(4)]
                    o07 = q[0] + q[1]
                    o815 = q[2] + q[3]
                    pref = [None, ones[0], t[0], t[0] + ones[2],
                            q[0], q[0] + ones[4], q[0] + t[2],
                            q[0] + (t[2] + ones[6]),
                            o07, o07 + ones[8], o07 + t[4],
                            o07 + (t[4] + ones[10]),
                            o07 + q[2], o07 + (q[2] + ones[12]),
                            o07 + (q[2] + t[6]),
                            o07 + (q[2] + (t[6] + ones[14]))]
                    for u in range(16):
                        c_u = cnt if u == 0 else cnt + pref[u]
                        pos = basev + jnp.minimum(c_u, SLOTS - 1)
                        idxv = iota16 + (jj * 256 + u * 16)
                        plsc.store_scatter(sraw, [pos], idxv, mask=masks[u])
                    return jnp.minimum(cnt + (o07 + o815), SLOTS - 1)

                cnt = lax.fori_loop(0, N // 256, chunks,
                                    jnp.zeros((16,), jnp.int32))
                for k in range(SLOTS):
                    slotpos = iota16 * SLOTS + (s * W + k)
                    idx_k = plsc.load_gather(sraw, [slotpos])
                    d_k = plsc.load_gather(rb, [idx_k + (s * N)])
                    filled = cnt > k
                    d_k = jnp.where(filled, d_k, infv)
                    scd[pl.ds(s * W + k * 16, 16)] = d_k
                    sci[pl.ds(s * W + k * 16, 16)] = idx_k
            pltpu.sync_copy(scd, cd_hbm.at[pl.ds((base + g * 8) * W, 8 * W)])
            pltpu.sync_copy(sci, ci_hbm.at[pl.ds((base + g * 8) * W, 8 * W)])

            @pl.when(g + 2 < GROUPS)
            def _(rb=rb, si=si, g=g):
                _fetch(g + 2, rb, si)
        return carry

    lax.fori_loop(0, GROUPS // 2, outer, jnp.int32(0))


def _sc_compact(d2, tau):
    mesh = plsc.VectorSubcoreMesh(core_axis_name="c", subcore_axis_name="s")
    f = functools.partial(
        pl.kernel,
        mesh=mesh,
        compiler_params=pltpu.CompilerParams(needs_layout_passes=False),
        out_type=[
            jax.ShapeDtypeStruct((ROWS * W,), jnp.float32),
            jax.ShapeDtypeStruct((ROWS * W,), jnp.int32),
        ],
        scratch_types=[
            pltpu.VMEM((8 * N,), jnp.float32),
            pltpu.VMEM((8 * N,), jnp.float32),
            pltpu.VMEM((8 * W,), jnp.float32),
            pltpu.VMEM((8 * W,), jnp.float32),
            pltpu.VMEM((8 * W,), jnp.int32),
            pltpu.VMEM((8 * W,), jnp.int32),
            pltpu.VMEM((8 * W,), jnp.int32),
            pltpu.VMEM((8 * W,), jnp.int32),
            pltpu.VMEM((RPW + 16,), jnp.float32),
            pltpu.SemaphoreType.DMA,
            pltpu.SemaphoreType.DMA,
        ],
    )(_sc_compact_body)
    return f(d2, tau)


RB2 = 512

def _finish_block(cd_ref, ci_ref, out_ref):
    d = cd_ref[...]
    idx = ci_ref[...]
    big = jnp.int32(N)
    inf = jnp.float32(jnp.inf)
    cols = []
    for _ in range(K):
        m = jnp.min(d, axis=1, keepdims=True)
        masked_idx = jnp.where(d == m, idx, big)
        pick = jnp.min(masked_idx, axis=1)
        cols.append(pick)
        d = jnp.where(idx == pick[:, None], inf, d)
    out_ref[...] = jnp.stack(cols, axis=1)


def kernel(xyz):
    d2, tau = pl.pallas_call(
        _dist_tau_block,
        grid=(B, N // QB),
        in_specs=[
            pl.BlockSpec((1, QB, 3), lambda i, j: (i, j, 0)),
            pl.BlockSpec((1, N, 3), lambda i, j: (i, 0, 0)),
        ],
        out_specs=[
            pl.BlockSpec((QB, N), lambda i, j: (i * (N // QB) + j, 0)),
            pl.BlockSpec((QB,), lambda i, j: (i * (N // QB) + j,)),
        ],
        out_shape=[
            jax.ShapeDtypeStruct((ROWS, N), jnp.float32),
            jax.ShapeDtypeStruct((ROWS,), jnp.float32),
        ],
    )(xyz, xyz)

    cd_flat, ci_flat = _sc_compact(d2, tau)
    cd = cd_flat.reshape(ROWS, W)
    ci = ci_flat.reshape(ROWS, W)

    out = pl.pallas_call(
        _finish_block,
        grid=(ROWS // RB2,),
        in_specs=[
            pl.BlockSpec((RB2, W), lambda i: (i, 0)),
            pl.BlockSpec((RB2, W), lambda i: (i, 0)),
        ],
        out_specs=pl.BlockSpec((RB2, K), lambda i: (i, 0)),
        out_shape=jax.ShapeDtypeStruct((ROWS, K), jnp.int32),
    )(cd, ci)
    return out.reshape(B, N, K).astype(jnp.int64)
